# baseline (speedup 1.0000x reference)
import functools
import math

import jax
import jax.numpy as jnp
import numpy as np
from jax.experimental import pallas as pl
from jax.experimental.pallas import tpu as pltpu

N = 4096
IN_DIM = 128
HID = 256
LAT = 128
DEPTH = 3
BPS = 2
RATIO = 0.5

ROWB = 128
NBLK = N // ROWB
LANE = 128
NROW = N // LANE


def _ln(v, g, b):
    m = jnp.mean(v, axis=-1, keepdims=True)
    var = jnp.mean((v - m) ** 2, axis=-1, keepdims=True)
    return (v - m) / jnp.sqrt(var + 1e-5) * g + b


def _silu(v):
    return v * jax.nn.sigmoid(v)


def _pre_kernel(x_ref, pos_ref, wi_ref, bi_ref, wp1_ref, bp1_ref, wp2_ref,
                bp2_ref, scale_ref, lng_ref, lnb_ref, h_ref, g_ref):
    h = jnp.dot(x_ref[...], wi_ref[...], preferred_element_type=jnp.float32)
    h = (h + bi_ref[...]) * scale_ref[0, 0]
    pe = _silu(jnp.dot(pos_ref[...], wp1_ref[...],
                       preferred_element_type=jnp.float32) + bp1_ref[...])
    pe = jnp.dot(pe, wp2_ref[...], preferred_element_type=jnp.float32) + bp2_ref[...]
    h = h + pe
    h_ref[...] = h
    g_ref[...] = _ln(h, lng_ref[...], lnb_ref[...])


def _enc_blk_kernel(xin_ref, g_full_ref, adj_ref, mask_ref, eps_ref,
                    w1_ref, b1_ref, w2_ref, b2_ref, nlng_ref, nlnb_ref,
                    pvec_ref, invn_ref,
                    h_ref, g_ref, score_ref, *, emit_score):
    i = pl.program_id(0)
    ns = jnp.dot(adj_ref[...], g_full_ref[...],
                 preferred_element_type=jnp.float32)
    g_blk = g_full_ref[pl.ds(i * ROWB, ROWB), :]
    h = (1.0 + eps_ref[0, 0]) * g_blk + ns
    h = _silu(jnp.dot(h, w1_ref[...], preferred_element_type=jnp.float32)
              + b1_ref[...])
    h = jnp.dot(h, w2_ref[...], preferred_element_type=jnp.float32) + b2_ref[...]
    h = xin_ref[...] + h
    h_ref[...] = h
    g_ref[...] = _ln(h, nlng_ref[...], nlnb_ref[...]) * mask_ref[...]
    if emit_score:
        score_ref[...] = jnp.dot(h, pvec_ref[...],
                                 preferred_element_type=jnp.float32) * invn_ref[0, 0]


def _sel_kernel(score_ref, mask_ref, sel_ref, nmask_ref, *, k, first):
    s_f = score_ref[...]
    bits = jax.lax.bitcast_convert_type(s_f, jnp.int32)
    s = jnp.where(bits >= 0, bits, bits ^ jnp.int32(0x7FFFFFFF))
    if first:
        valid = jnp.ones(s.shape, jnp.bool_)
    else:
        valid = mask_ref[...] > 0.0

    def cnt_ge(t):
        return jnp.sum(jnp.where(valid & (s >= t), 1, 0))

    int_min = jnp.int32(-2147483648)
    int_max = jnp.int32(2147483647)

    def t_body(_, carry):
        lo, hi = carry
        mid = (lo & hi) + ((lo ^ hi) >> 1)
        mid = jnp.maximum(mid, lo + 1)
        go = cnt_ge(mid) >= k
        return (jnp.where(go, mid, lo), jnp.where(go, hi, mid - 1))

    lo0 = jnp.where(cnt_ge(int_max) >= k, int_max, int_min)
    lo, _ = jax.lax.fori_loop(0, 32, t_body, (lo0, int_max))
    t = lo
    m = k - jnp.sum(jnp.where(valid & (s > t), 1, 0))

    idx = (jax.lax.broadcasted_iota(jnp.int32, s.shape, 0) * LANE
           + jax.lax.broadcasted_iota(jnp.int32, s.shape, 1))
    eq = valid & (s == t)

    def j_body(_, carry):
        lo_j, hi_j = carry
        mid = (lo_j + hi_j) >> 1
        c = jnp.sum(jnp.where(eq & (idx <= mid), 1, 0))
        go = c >= m
        return (jnp.where(go, lo_j, mid + 1), jnp.where(go, mid, hi_j))

    lo_j, _ = jax.lax.fori_loop(0, 13, j_body, (jnp.int32(0), jnp.int32(N - 1)))
    keep = valid & ((s > t) | (eq & (idx <= lo_j) & (m > 0)))
    sel_ref[...] = jnp.where(keep, jnp.tanh(s_f), 0.0)
    nmask_ref[...] = jnp.where(keep, 1.0, 0.0)


def _apply_kernel(h_ref, sel_ref, mask_ref, lng_ref, lnb_ref, h_out_ref,
                  g_ref):
    h = h_ref[...] * sel_ref[...]
    h_out_ref[...] = h
    g_ref[...] = _ln(h, lng_ref[...], lnb_ref[...]) * mask_ref[...]


def _latent_kernel(h_ref, mask_ref, wtl_ref, btl_ref, wfl_ref, bfl_ref,
                   z_ref, hd_ref, *, n_kept):
    hg = jnp.sum(h_ref[...] * mask_ref[...], axis=0, keepdims=True) / n_kept
    z = jnp.dot(hg, wtl_ref[...], preferred_element_type=jnp.float32) + btl_ref[...]
    z_ref[...] = z
    hd_ref[...] = jnp.dot(z, wfl_ref[...],
                          preferred_element_type=jnp.float32) + bfl_ref[...]


def _dec_kernel(m_ref, hd16_ref, lngs_ref, lnbs_ref, w1s_ref, b1s_ref,
                w2s_ref, b2s_ref, wo1_ref, bo1_ref, wo2_ref, bo2_ref,
                out_ref, *, n_dec):
    h = jnp.dot(m_ref[...], hd16_ref[...], preferred_element_type=jnp.float32)
    for i in range(n_dec):
        t = _ln(h, lngs_ref[i], lnbs_ref[i])
        t = _silu(jnp.dot(t, w1s_ref[i], preferred_element_type=jnp.float32)
                  + b1s_ref[i])
        t = jnp.dot(t, w2s_ref[i], preferred_element_type=jnp.float32) + b2s_ref[i]
        h = h + t
    o = _silu(jnp.dot(h, wo1_ref[...], preferred_element_type=jnp.float32)
              + bo1_ref[...])
    out_ref[...] = jnp.dot(o, wo2_ref[...],
                           preferred_element_type=jnp.float32) + bo2_ref[...]


def _row(b):
    return b.reshape(1, -1)


def _full(shape):
    nd = len(shape)
    return pl.BlockSpec(shape, lambda i, _nd=nd: (0,) * _nd)


def _smem11():
    return pl.BlockSpec(memory_space=pltpu.SMEM)


def _enc_block_call(xin, g_full, adj, mask_col, bp, next_ln, pvec_c, invn,
                    emit_score):
    grid = (NBLK,)
    kern = functools.partial(_enc_blk_kernel, emit_score=emit_score)
    out_shapes = [
        jax.ShapeDtypeStruct((N, HID), jnp.float32),
        jax.ShapeDtypeStruct((N, HID), jnp.float32),
        jax.ShapeDtypeStruct((N, 1), jnp.float32),
    ]
    out_specs = [
        pl.BlockSpec((ROWB, HID), lambda i: (i, 0)),
        pl.BlockSpec((ROWB, HID), lambda i: (i, 0)),
        pl.BlockSpec((ROWB, 1), lambda i: (i, 0)),
    ]
    h, g, score = pl.pallas_call(
        kern,
        grid=grid,
        in_specs=[
            pl.BlockSpec((ROWB, HID), lambda i: (i, 0)),
            _full((N, HID)),
            pl.BlockSpec((ROWB, N), lambda i: (i, 0)),
            pl.BlockSpec((ROWB, 1), lambda i: (i, 0)),
            _smem11(),
            _full((HID, HID)), _full((1, HID)),
            _full((HID, HID)), _full((1, HID)),
            _full((1, HID)), _full((1, HID)),
            _full((HID, 1)),
            _smem11(),
        ],
        out_shape=out_shapes,
        out_specs=out_specs,
    )(xin, g_full, adj, mask_col, bp["eps"].reshape(1, 1),
      bp["lin1"]["w"], _row(bp["lin1"]["b"]),
      bp["lin2"]["w"], _row(bp["lin2"]["b"]),
      _row(next_ln[0]), _row(next_ln[1]), pvec_c, invn)
    return h, g, score


def kernel(x, adj, pos, batch_size, params):
    f32 = jnp.float32
    B_static = x.shape[0] // adj.shape[0]
    scale = (jnp.asarray(batch_size, f32) / B_static).reshape(1, 1)
    adj0 = adj.astype(f32)

    pos_p = jnp.pad(pos.astype(f32), ((0, 0), (0, 5)))
    wp1 = jnp.pad(params["pos_mlp"][0]["w"].astype(f32), ((0, 5), (0, 0)))

    enc_seq = []
    for d in range(DEPTH):
        for b in range(BPS):
            enc_seq.append(("blk", d, b))
        enc_seq.append(("pool", d))
    for b in range(BPS):
        enc_seq.append(("fblk", b))

    def blk_params(tag):
        if tag[0] == "blk":
            return params["enc"][tag[1]][tag[2]]
        return params["final_enc"][tag[1]]

    def next_ln_of(i):
        for tag in enc_seq[i + 1:]:
            if tag[0] in ("blk", "fblk"):
                p = blk_params(tag)
                return (p["ln_g"], p["ln_b"])
        p = params["final_enc"][-1]
        return (p["ln_g"], p["ln_b"])

    ln0 = params["enc"][0][0]
    h0, g0 = pl.pallas_call(
        _pre_kernel,
        grid=(NBLK,),
        in_specs=[
            pl.BlockSpec((ROWB, IN_DIM), lambda i: (i, 0)),
            pl.BlockSpec((ROWB, 8), lambda i: (i, 0)),
            _full((IN_DIM, HID)), _full((1, HID)),
            _full((8, HID)), _full((1, HID)),
            _full((HID, HID)), _full((1, HID)),
            _smem11(),
            _full((1, HID)), _full((1, HID)),
        ],
        out_shape=[jax.ShapeDtypeStruct((N, HID), f32),
                   jax.ShapeDtypeStruct((N, HID), f32)],
        out_specs=[pl.BlockSpec((ROWB, HID), lambda i: (i, 0)),
                   pl.BlockSpec((ROWB, HID), lambda i: (i, 0))],
    )(x.astype(f32), pos_p, params["in_proj"]["w"], _row(params["in_proj"]["b"]),
      wp1, _row(params["pos_mlp"][0]["b"]),
      params["pos_mlp"][1]["w"], _row(params["pos_mlp"][1]["b"]),
      scale, _row(ln0["ln_g"]), _row(ln0["ln_b"]))

    mask_col = jnp.ones((N, 1), f32)
    h_cur, g_cur = h0, g0
    npg = N
    for i, tag in enumerate(enc_seq):
        if tag[0] in ("blk", "fblk"):
            bp = blk_params(tag)
            nxt = enc_seq[i + 1] if i + 1 < len(enc_seq) else None
            emit_score = nxt is not None and nxt[0] == "pool"
            if emit_score:
                d = nxt[1]
                pv = params["pools"][d].astype(f32)
                invn = (1.0 / (jnp.sqrt(jnp.sum(pv * pv)) + 1e-12)).reshape(1, 1)
                pv_c = pv.reshape(HID, 1)
            else:
                pv_c = jnp.zeros((HID, 1), f32)
                invn = jnp.zeros((1, 1), f32)
            h_cur, g_cur, score = _enc_block_call(
                h_cur, g_cur, adj0, mask_col, bp, next_ln_of(i), pv_c, invn,
                emit_score)
        else:
            d = tag[1]
            k = int(math.ceil(RATIO * npg))
            sel, nmask = pl.pallas_call(
                functools.partial(_sel_kernel, k=k, first=(d == 0)),
                grid=(1,),
                in_specs=[_full((NROW, LANE)), _full((NROW, LANE))],
                out_shape=[jax.ShapeDtypeStruct((NROW, LANE), f32),
                           jax.ShapeDtypeStruct((NROW, LANE), f32)],
                out_specs=[_full((NROW, LANE)), _full((NROW, LANE))],
            )(score.reshape(NROW, LANE), mask_col.reshape(NROW, LANE))
            sel_col = sel.reshape(N, 1)
            mask_col = nmask.reshape(N, 1)
            nln = next_ln_of(i)
            h_cur, g_cur = pl.pallas_call(
                _apply_kernel,
                grid=(NBLK,),
                in_specs=[
                    pl.BlockSpec((ROWB, HID), lambda i: (i, 0)),
                    pl.BlockSpec((ROWB, 1), lambda i: (i, 0)),
                    pl.BlockSpec((ROWB, 1), lambda i: (i, 0)),
                    _full((1, HID)), _full((1, HID)),
                ],
                out_shape=[jax.ShapeDtypeStruct((N, HID), f32),
                           jax.ShapeDtypeStruct((N, HID), f32)],
                out_specs=[pl.BlockSpec((ROWB, HID), lambda i: (i, 0)),
                           pl.BlockSpec((ROWB, HID), lambda i: (i, 0))],
            )(h_cur, sel_col, mask_col, _row(nln[0]), _row(nln[1]))
            npg = k

    z, hd = pl.pallas_call(
        functools.partial(_latent_kernel, n_kept=float(npg)),
        grid=(1,),
        in_specs=[_full((N, HID)), _full((N, 1)),
                  _full((HID, LAT)), _full((1, LAT)),
                  _full((LAT, 16 * HID)), _full((1, 16 * HID))],
        out_shape=[jax.ShapeDtypeStruct((1, LAT), f32),
                   jax.ShapeDtypeStruct((1, 16 * HID), f32)],
        out_specs=[_full((1, LAT)), _full((1, 16 * HID))],
    )(h_cur, mask_col, params["to_latent"]["w"], _row(params["to_latent"]["b"]),
      params["from_latent"]["w"], _row(params["from_latent"]["b"]))

    hd16 = hd.reshape(16, HID)

    src = (np.arange(N) + 0.5) * (16.0 / N) - 0.5
    src = np.clip(src, 0.0, 15.0)
    lo = np.floor(src).astype(np.int32)
    hi = np.minimum(lo + 1, 15)
    w = (src - lo).astype(np.float32)
    m_np = np.zeros((N, 16), np.float32)
    m_np[np.arange(N), lo] += 1.0 - w
    m_np[np.arange(N), hi] += w
    m_interp = jnp.asarray(m_np)

    n_dec = DEPTH * BPS
    dlngs = jnp.stack([p["ln_g"] for p in params["dec"]]).reshape(n_dec, 1, HID)
    dlnbs = jnp.stack([p["ln_b"] for p in params["dec"]]).reshape(n_dec, 1, HID)
    dw1 = jnp.stack([p["lin1"]["w"] for p in params["dec"]])
    db1 = jnp.stack([p["lin1"]["b"] for p in params["dec"]]).reshape(n_dec, 1, HID)
    dw2 = jnp.stack([p["lin2"]["w"] for p in params["dec"]])
    db2 = jnp.stack([p["lin2"]["b"] for p in params["dec"]]).reshape(n_dec, 1, HID)

    out = pl.pallas_call(
        functools.partial(_dec_kernel, n_dec=n_dec),
        grid=(NBLK,),
        in_specs=[
            pl.BlockSpec((ROWB, 16), lambda i: (i, 0)),
            _full((16, HID)),
            _full((n_dec, 1, HID)), _full((n_dec, 1, HID)),
            _full((n_dec, HID, HID)), _full((n_dec, 1, HID)),
            _full((n_dec, HID, HID)), _full((n_dec, 1, HID)),
            _full((HID, HID)), _full((1, HID)),
            _full((HID, IN_DIM)), _full((1, IN_DIM)),
        ],
        out_shape=jax.ShapeDtypeStruct((N, IN_DIM), f32),
        out_specs=pl.BlockSpec((ROWB, IN_DIM), lambda i: (i, 0)),
    )(m_interp, hd16, dlngs, dlnbs, dw1, db1, dw2, db2,
      params["out_proj"][0]["w"], _row(params["out_proj"][0]["b"]),
      params["out_proj"][1]["w"], _row(params["out_proj"][1]["b"]))

    return out.reshape(B_static, N, IN_DIM), z


# speedup vs baseline: 1.3257x; 1.3257x over previous
import functools
import math

import jax
import jax.numpy as jnp
import numpy as np
from jax.experimental import pallas as pl
from jax.experimental.pallas import tpu as pltpu

N = 4096
IN_DIM = 128
HID = 256
LAT = 128
DEPTH = 3
BPS = 2
RATIO = 0.5

ROWB = 256
NBLK = N // ROWB
LANE = 128
NROW = N // LANE


def _ln(v, g, b):
    m = jnp.mean(v, axis=-1, keepdims=True)
    var = jnp.mean((v - m) ** 2, axis=-1, keepdims=True)
    return (v - m) / jnp.sqrt(var + 1e-5) * g + b


def _silu(v):
    return v * jax.nn.sigmoid(v)


def _split_bf16(g):
    hi = g.astype(jnp.bfloat16)
    lo = (g - hi.astype(jnp.float32)).astype(jnp.bfloat16)
    return hi, lo


def _pre_kernel(x_ref, pos_ref, wi_ref, bi_ref, wp1_ref, bp1_ref, wp2_ref,
                bp2_ref, scale_ref, lng_ref, lnb_ref, h_ref, ghi_ref, glo_ref):
    h = jnp.dot(x_ref[...], wi_ref[...], preferred_element_type=jnp.float32)
    h = (h + bi_ref[...]) * scale_ref[0, 0]
    pe = _silu(jnp.dot(pos_ref[...], wp1_ref[...],
                       preferred_element_type=jnp.float32) + bp1_ref[...])
    pe = jnp.dot(pe, wp2_ref[...], preferred_element_type=jnp.float32) + bp2_ref[...]
    h = h + pe
    h_ref[...] = h
    ghi_ref[...], glo_ref[...] = _split_bf16(_ln(h, lng_ref[...], lnb_ref[...]))


def _enc_blk_kernel(xin_ref, ghi_full_ref, glo_full_ref, adj_ref, mask_ref,
                    eps_ref, w1_ref, b1_ref, w2_ref, b2_ref, nlng_ref,
                    nlnb_ref, pvec_ref, invn_ref,
                    h_ref, ghi_ref, glo_ref, score_ref, *, emit_score):
    i = pl.program_id(0)
    a = adj_ref[...]
    ns = (jnp.dot(a, ghi_full_ref[...], preferred_element_type=jnp.float32)
          + jnp.dot(a, glo_full_ref[...], preferred_element_type=jnp.float32))
    g_blk = (ghi_full_ref[pl.ds(i * ROWB, ROWB), :].astype(jnp.float32)
             + glo_full_ref[pl.ds(i * ROWB, ROWB), :].astype(jnp.float32))
    h = (1.0 + eps_ref[0, 0]) * g_blk + ns
    h = _silu(jnp.dot(h, w1_ref[...], preferred_element_type=jnp.float32)
              + b1_ref[...])
    h = jnp.dot(h, w2_ref[...], preferred_element_type=jnp.float32) + b2_ref[...]
    h = xin_ref[...] + h
    h_ref[...] = h
    g = _ln(h, nlng_ref[...], nlnb_ref[...]) * mask_ref[...]
    ghi_ref[...], glo_ref[...] = _split_bf16(g)
    if emit_score:
        score_ref[...] = jnp.dot(h, pvec_ref[...],
                                 preferred_element_type=jnp.float32) * invn_ref[0, 0]


def _sel_kernel(score_ref, mask_ref, sel_ref, nmask_ref, *, k, first):
    s_f = score_ref[...]
    bits = jax.lax.bitcast_convert_type(s_f, jnp.int32)
    s = jnp.where(bits >= 0, bits, bits ^ jnp.int32(0x7FFFFFFF))
    if first:
        valid = jnp.ones(s.shape, jnp.bool_)
    else:
        valid = mask_ref[...] > 0.0

    def cnt_ge(t):
        return jnp.sum(jnp.where(valid & (s >= t), 1, 0))

    int_min = jnp.int32(-2147483648)
    int_max = jnp.int32(2147483647)

    def t_body(_, carry):
        lo, hi = carry
        mid = (lo & hi) + ((lo ^ hi) >> 1)
        mid = jnp.maximum(mid, lo + 1)
        go = cnt_ge(mid) >= k
        return (jnp.where(go, mid, lo), jnp.where(go, hi, mid - 1))

    lo0 = jnp.where(cnt_ge(int_max) >= k, int_max, int_min)
    lo, _ = jax.lax.fori_loop(0, 32, t_body, (lo0, int_max))
    t = lo
    m = k - jnp.sum(jnp.where(valid & (s > t), 1, 0))

    idx = (jax.lax.broadcasted_iota(jnp.int32, s.shape, 0) * LANE
           + jax.lax.broadcasted_iota(jnp.int32, s.shape, 1))
    eq = valid & (s == t)

    def j_body(_, carry):
        lo_j, hi_j = carry
        mid = (lo_j + hi_j) >> 1
        c = jnp.sum(jnp.where(eq & (idx <= mid), 1, 0))
        go = c >= m
        return (jnp.where(go, lo_j, mid + 1), jnp.where(go, mid, hi_j))

    lo_j, _ = jax.lax.fori_loop(0, 13, j_body, (jnp.int32(0), jnp.int32(N - 1)))
    keep = valid & ((s > t) | (eq & (idx <= lo_j) & (m > 0)))
    sel_ref[...] = jnp.where(keep, jnp.tanh(s_f), 0.0)
    nmask_ref[...] = jnp.where(keep, 1.0, 0.0)


def _apply_kernel(h_ref, sel_ref, mask_ref, lng_ref, lnb_ref, h_out_ref,
                  ghi_ref, glo_ref):
    h = h_ref[...] * sel_ref[...]
    h_out_ref[...] = h
    g = _ln(h, lng_ref[...], lnb_ref[...]) * mask_ref[...]
    ghi_ref[...], glo_ref[...] = _split_bf16(g)


def _latent_kernel(h_ref, mask_ref, wtl_ref, btl_ref, wfl_ref, bfl_ref,
                   z_ref, hd_ref, *, n_kept):
    hg = jnp.sum(h_ref[...] * mask_ref[...], axis=0, keepdims=True) / n_kept
    z = jnp.dot(hg, wtl_ref[...], preferred_element_type=jnp.float32) + btl_ref[...]
    z_ref[...] = z
    hd_ref[...] = jnp.dot(z, wfl_ref[...],
                          preferred_element_type=jnp.float32) + bfl_ref[...]


def _dec_kernel(m_ref, hd16_ref, lngs_ref, lnbs_ref, w1s_ref, b1s_ref,
                w2s_ref, b2s_ref, wo1_ref, bo1_ref, wo2_ref, bo2_ref,
                out_ref, *, n_dec):
    h = jnp.dot(m_ref[...], hd16_ref[...], preferred_element_type=jnp.float32)
    for i in range(n_dec):
        t = _ln(h, lngs_ref[i], lnbs_ref[i])
        t = _silu(jnp.dot(t, w1s_ref[i], preferred_element_type=jnp.float32)
                  + b1s_ref[i])
        t = jnp.dot(t, w2s_ref[i], preferred_element_type=jnp.float32) + b2s_ref[i]
        h = h + t
    o = _silu(jnp.dot(h, wo1_ref[...], preferred_element_type=jnp.float32)
              + bo1_ref[...])
    out_ref[...] = jnp.dot(o, wo2_ref[...],
                           preferred_element_type=jnp.float32) + bo2_ref[...]


def _row(b):
    return b.reshape(1, -1)


def _full(shape):
    nd = len(shape)
    return pl.BlockSpec(shape, lambda i, _nd=nd: (0,) * _nd)


def _smem11():
    return pl.BlockSpec(memory_space=pltpu.SMEM)


def _enc_block_call(xin, ghi, glo, adj_bf, mask_col, bp, next_ln, pvec_c,
                    invn, emit_score):
    kern = functools.partial(_enc_blk_kernel, emit_score=emit_score)
    out_shapes = [
        jax.ShapeDtypeStruct((N, HID), jnp.float32),
        jax.ShapeDtypeStruct((N, HID), jnp.bfloat16),
        jax.ShapeDtypeStruct((N, HID), jnp.bfloat16),
        jax.ShapeDtypeStruct((N, 1), jnp.float32),
    ]
    out_specs = [
        pl.BlockSpec((ROWB, HID), lambda i: (i, 0)),
        pl.BlockSpec((ROWB, HID), lambda i: (i, 0)),
        pl.BlockSpec((ROWB, HID), lambda i: (i, 0)),
        pl.BlockSpec((ROWB, 1), lambda i: (i, 0)),
    ]
    h, nghi, nglo, score = pl.pallas_call(
        kern,
        grid=(NBLK,),
        in_specs=[
            pl.BlockSpec((ROWB, HID), lambda i: (i, 0)),
            _full((N, HID)),
            _full((N, HID)),
            pl.BlockSpec((ROWB, N), lambda i: (i, 0)),
            pl.BlockSpec((ROWB, 1), lambda i: (i, 0)),
            _smem11(),
            _full((HID, HID)), _full((1, HID)),
            _full((HID, HID)), _full((1, HID)),
            _full((1, HID)), _full((1, HID)),
            _full((HID, 1)),
            _smem11(),
        ],
        out_shape=out_shapes,
        out_specs=out_specs,
    )(xin, ghi, glo, adj_bf, mask_col, bp["eps"].reshape(1, 1),
      bp["lin1"]["w"], _row(bp["lin1"]["b"]),
      bp["lin2"]["w"], _row(bp["lin2"]["b"]),
      _row(next_ln[0]), _row(next_ln[1]), pvec_c, invn)
    return h, nghi, nglo, score


def kernel(x, adj, pos, batch_size, params):
    f32 = jnp.float32
    B_static = x.shape[0] // adj.shape[0]
    scale = (jnp.asarray(batch_size, f32) / B_static).reshape(1, 1)
    adj_bf = adj.astype(jnp.bfloat16)

    pos_p = jnp.pad(pos.astype(f32), ((0, 0), (0, 5)))
    wp1 = jnp.pad(params["pos_mlp"][0]["w"].astype(f32), ((0, 5), (0, 0)))

    enc_seq = []
    for d in range(DEPTH):
        for b in range(BPS):
            enc_seq.append(("blk", d, b))
        enc_seq.append(("pool", d))
    for b in range(BPS):
        enc_seq.append(("fblk", b))

    def blk_params(tag):
        if tag[0] == "blk":
            return params["enc"][tag[1]][tag[2]]
        return params["final_enc"][tag[1]]

    def next_ln_of(i):
        for tag in enc_seq[i + 1:]:
            if tag[0] in ("blk", "fblk"):
                p = blk_params(tag)
                return (p["ln_g"], p["ln_b"])
        p = params["final_enc"][-1]
        return (p["ln_g"], p["ln_b"])

    ln0 = params["enc"][0][0]
    h0, ghi0, glo0 = pl.pallas_call(
        _pre_kernel,
        grid=(NBLK,),
        in_specs=[
            pl.BlockSpec((ROWB, IN_DIM), lambda i: (i, 0)),
            pl.BlockSpec((ROWB, 8), lambda i: (i, 0)),
            _full((IN_DIM, HID)), _full((1, HID)),
            _full((8, HID)), _full((1, HID)),
            _full((HID, HID)), _full((1, HID)),
            _smem11(),
            _full((1, HID)), _full((1, HID)),
        ],
        out_shape=[jax.ShapeDtypeStruct((N, HID), f32),
                   jax.ShapeDtypeStruct((N, HID), jnp.bfloat16),
                   jax.ShapeDtypeStruct((N, HID), jnp.bfloat16)],
        out_specs=[pl.BlockSpec((ROWB, HID), lambda i: (i, 0)),
                   pl.BlockSpec((ROWB, HID), lambda i: (i, 0)),
                   pl.BlockSpec((ROWB, HID), lambda i: (i, 0))],
    )(x.astype(f32), pos_p, params["in_proj"]["w"], _row(params["in_proj"]["b"]),
      wp1, _row(params["pos_mlp"][0]["b"]),
      params["pos_mlp"][1]["w"], _row(params["pos_mlp"][1]["b"]),
      scale, _row(ln0["ln_g"]), _row(ln0["ln_b"]))

    mask_col = jnp.ones((N, 1), f32)
    h_cur, ghi_cur, glo_cur = h0, ghi0, glo0
    npg = N
    for i, tag in enumerate(enc_seq):
        if tag[0] in ("blk", "fblk"):
            bp = blk_params(tag)
            nxt = enc_seq[i + 1] if i + 1 < len(enc_seq) else None
            emit_score = nxt is not None and nxt[0] == "pool"
            if emit_score:
                d = nxt[1]
                pv = params["pools"][d].astype(f32)
                invn = (1.0 / (jnp.sqrt(jnp.sum(pv * pv)) + 1e-12)).reshape(1, 1)
                pv_c = pv.reshape(HID, 1)
            else:
                pv_c = jnp.zeros((HID, 1), f32)
                invn = jnp.zeros((1, 1), f32)
            h_cur, ghi_cur, glo_cur, score = _enc_block_call(
                h_cur, ghi_cur, glo_cur, adj_bf, mask_col, bp, next_ln_of(i),
                pv_c, invn, emit_score)
        else:
            d = tag[1]
            k = int(math.ceil(RATIO * npg))
            sel, nmask = pl.pallas_call(
                functools.partial(_sel_kernel, k=k, first=(d == 0)),
                grid=(1,),
                in_specs=[_full((NROW, LANE)), _full((NROW, LANE))],
                out_shape=[jax.ShapeDtypeStruct((NROW, LANE), f32),
                           jax.ShapeDtypeStruct((NROW, LANE), f32)],
                out_specs=[_full((NROW, LANE)), _full((NROW, LANE))],
            )(score.reshape(NROW, LANE), mask_col.reshape(NROW, LANE))
            sel_col = sel.reshape(N, 1)
            mask_col = nmask.reshape(N, 1)
            nln = next_ln_of(i)
            h_cur, ghi_cur, glo_cur = pl.pallas_call(
                _apply_kernel,
                grid=(NBLK,),
                in_specs=[
                    pl.BlockSpec((ROWB, HID), lambda i: (i, 0)),
                    pl.BlockSpec((ROWB, 1), lambda i: (i, 0)),
                    pl.BlockSpec((ROWB, 1), lambda i: (i, 0)),
                    _full((1, HID)), _full((1, HID)),
                ],
                out_shape=[jax.ShapeDtypeStruct((N, HID), f32),
                           jax.ShapeDtypeStruct((N, HID), jnp.bfloat16),
                           jax.ShapeDtypeStruct((N, HID), jnp.bfloat16)],
                out_specs=[pl.BlockSpec((ROWB, HID), lambda i: (i, 0)),
                           pl.BlockSpec((ROWB, HID), lambda i: (i, 0)),
                           pl.BlockSpec((ROWB, HID), lambda i: (i, 0))],
            )(h_cur, sel_col, mask_col, _row(nln[0]), _row(nln[1]))
            npg = k

    z, hd = pl.pallas_call(
        functools.partial(_latent_kernel, n_kept=float(npg)),
        grid=(1,),
        in_specs=[_full((N, HID)), _full((N, 1)),
                  _full((HID, LAT)), _full((1, LAT)),
                  _full((LAT, 16 * HID)), _full((1, 16 * HID))],
        out_shape=[jax.ShapeDtypeStruct((1, LAT), f32),
                   jax.ShapeDtypeStruct((1, 16 * HID), f32)],
        out_specs=[_full((1, LAT)), _full((1, 16 * HID))],
    )(h_cur, mask_col, params["to_latent"]["w"], _row(params["to_latent"]["b"]),
      params["from_latent"]["w"], _row(params["from_latent"]["b"]))

    hd16 = hd.reshape(16, HID)

    src = (np.arange(N) + 0.5) * (16.0 / N) - 0.5
    src = np.clip(src, 0.0, 15.0)
    lo = np.floor(src).astype(np.int32)
    hi = np.minimum(lo + 1, 15)
    w = (src - lo).astype(np.float32)
    m_np = np.zeros((N, 16), np.float32)
    m_np[np.arange(N), lo] += 1.0 - w
    m_np[np.arange(N), hi] += w
    m_interp = jnp.asarray(m_np)

    n_dec = DEPTH * BPS
    dlngs = jnp.stack([p["ln_g"] for p in params["dec"]]).reshape(n_dec, 1, HID)
    dlnbs = jnp.stack([p["ln_b"] for p in params["dec"]]).reshape(n_dec, 1, HID)
    dw1 = jnp.stack([p["lin1"]["w"] for p in params["dec"]])
    db1 = jnp.stack([p["lin1"]["b"] for p in params["dec"]]).reshape(n_dec, 1, HID)
    dw2 = jnp.stack([p["lin2"]["w"] for p in params["dec"]])
    db2 = jnp.stack([p["lin2"]["b"] for p in params["dec"]]).reshape(n_dec, 1, HID)

    out = pl.pallas_call(
        functools.partial(_dec_kernel, n_dec=n_dec),
        grid=(NBLK,),
        in_specs=[
            pl.BlockSpec((ROWB, 16), lambda i: (i, 0)),
            _full((16, HID)),
            _full((n_dec, 1, HID)), _full((n_dec, 1, HID)),
            _full((n_dec, HID, HID)), _full((n_dec, 1, HID)),
            _full((n_dec, HID, HID)), _full((n_dec, 1, HID)),
            _full((HID, HID)), _full((1, HID)),
            _full((HID, IN_DIM)), _full((1, IN_DIM)),
        ],
        out_shape=jax.ShapeDtypeStruct((N, IN_DIM), f32),
        out_specs=pl.BlockSpec((ROWB, IN_DIM), lambda i: (i, 0)),
    )(m_interp, hd16, dlngs, dlnbs, dw1, db1, dw2, db2,
      params["out_proj"][0]["w"], _row(params["out_proj"][0]["b"]),
      params["out_proj"][1]["w"], _row(params["out_proj"][1]["b"]))

    return out.reshape(B_static, N, IN_DIM), z
